# baseline (speedup 1.0000x reference)
import jax
import jax.numpy as jnp
from jax import lax
from jax.experimental import pallas as pl
from jax.experimental.pallas import tpu as pltpu
from jax.experimental.pallas import tpu_sc as plsc

N = 10000
D = 128
E = 160000
NS = 16
CH = 80
CHV = 40
GC = 25
NG = 5
FB = 8
CW = 16
NK = D // 16


def _body(x_user, x_item, src_a, dst_a, src_r, dst_r, src_v, dst_v,
          out_user, buys_mean, pacc, pcnt,
          acc, cnt, rows0, rows1, vrows0, vrows1,
          src_i, dst_i, vsrc_i, vdst_i, fcnt, ones, vones,
          sem0, sem1, ssem0, ssem1, osem):
    core = lax.axis_index("c")
    sub = lax.axis_index("s")
    fbase = sub * (FB * CH)

    zero16 = jnp.zeros((16,), jnp.float32)
    one16 = jnp.ones((16,), jnp.float32)

    def init_ones(r, carry):
        ones[r, pl.ds(0, CW)] = one16
        return carry

    lax.fori_loop(0, CH, init_ones, 0)

    def init_vones(r, carry):
        vones[r, pl.ds(0, CW)] = one16
        return carry

    lax.fori_loop(0, CHV, init_vones, 0)

    def conv_loop(x_hbm, src_hbm, dst_hbm, bufs, sbuf, dbuf, onesbuf,
                  core_split):
        gsems = (sem0, sem1)
        ssems = (ssem0, ssem1)

        for g in range(NG):
            if core_split:
                pltpu.sync_copy(src_hbm.at[core, sub, g], sbuf)
                pltpu.sync_copy(dst_hbm.at[core, sub, g], dbuf)
            else:
                pltpu.sync_copy(src_hbm.at[sub, g], sbuf)
                pltpu.sync_copy(dst_hbm.at[sub, g], dbuf)

            def gather(cc, b):
                return pltpu.make_async_copy(
                    x_hbm.at[sbuf.at[cc]], bufs[b], gsems[b])

            def scat(cc, b):
                return pltpu.make_async_copy(
                    bufs[b], acc.at[dbuf.at[cc]], ssems[b])

            def onescat(cc):
                return pltpu.make_async_copy(
                    onesbuf, cnt.at[dbuf.at[cc]], osem)

            def start_chunk(cc, b):
                gather(cc, b).wait()
                pltpu.async_copy(bufs[b], acc.at[dbuf.at[cc]], ssems[b],
                                 add=True)
                pltpu.async_copy(onesbuf, cnt.at[dbuf.at[cc]], osem,
                                 add=True)

            gather(0, 0).start()
            start_chunk(0, 0)
            gather(1, 1).start()

            def pair(i, carry):
                c_odd = 2 * i + 1
                start_chunk(c_odd, 1)
                scat(c_odd - 1, 0).wait()
                gather(c_odd + 1, 0).start()
                c_even = 2 * i + 2
                start_chunk(c_even, 0)
                scat(c_even - 1, 1).wait()
                gather(c_even + 1, 1).start()
                return carry

            lax.fori_loop(0, GC // 2 - 1, pair, 0)

            start_chunk(GC - 2, 1)
            scat(GC - 3, 0).wait()
            gather(GC - 1, 0).start()
            start_chunk(GC - 1, 0)
            scat(GC - 2, 1).wait()
            scat(GC - 1, 0).wait()

            def drain_ones(i, carry):
                onescat(0).wait()
                return carry

            lax.fori_loop(0, GC, drain_ones, 0)

    def zero_slices():
        def zrow(r, carry):
            for k in range(NK):
                rows0[r, pl.ds(k * 16, 16)] = zero16
            fcnt[r, pl.ds(0, CW)] = zero16
            return carry

        lax.fori_loop(0, CH, zrow, 0)
        for c in range(FB):
            rowbase = fbase + c * CH

            @pl.when(rowbase < N)
            def _():
                pltpu.sync_copy(rows0, acc.at[pl.ds(rowbase, CH)])
                pltpu.sync_copy(fcnt, cnt.at[pl.ds(rowbase, CH)])

    def finalize(out_hbm):
        for c in range(FB):
            rowbase = fbase + c * CH

            @pl.when(rowbase < N)
            def _():
                pltpu.sync_copy(acc.at[pl.ds(rowbase, CH)], rows0)
                pltpu.sync_copy(cnt.at[pl.ds(rowbase, CH)], fcnt)

                def row_fn(r, carry):
                    cv = fcnt[r, pl.ds(0, CW)]
                    scale = 1.0 / jnp.maximum(cv, 1.0)
                    for k in range(NK):
                        rows0[r, pl.ds(k * 16, 16)] = (
                            rows0[r, pl.ds(k * 16, 16)] * scale)
                    return carry

                lax.fori_loop(0, CH, row_fn, 0)
                pltpu.sync_copy(rows0, out_hbm.at[pl.ds(rowbase, CH)])

    def dump_partials():
        for c in range(FB):
            rowbase = fbase + c * CH

            @pl.when(rowbase < N)
            def _():
                pltpu.sync_copy(acc.at[pl.ds(rowbase, CH)], rows0)
                pltpu.sync_copy(rows0, pacc.at[core, pl.ds(rowbase, CH)])
                pltpu.sync_copy(cnt.at[pl.ds(rowbase, CH)], fcnt)
                pltpu.sync_copy(fcnt, pcnt.at[core, pl.ds(rowbase, CH)])

    zero_slices()
    plsc.subcore_barrier()

    @pl.when(core == 0)
    def _():
        conv_loop(x_user, src_a, dst_a, (rows0, rows1), src_i, dst_i,
                  ones, False)

    @pl.when(core == 1)
    def _():
        conv_loop(x_item, src_r, dst_r, (rows0, rows1), src_i, dst_i,
                  ones, False)

    plsc.subcore_barrier()

    @pl.when(core == 0)
    def _():
        finalize(buys_mean)

    @pl.when(core == 1)
    def _():
        finalize(out_user)

    zero_slices()
    plsc.subcore_barrier()
    conv_loop(x_user, src_v, dst_v, (vrows0, vrows1), vsrc_i, vdst_i,
              vones, True)
    plsc.subcore_barrier()
    dump_partials()


def _combine_body(bm_ref, pacc_ref, pcnt_ref, out_ref):
    s = pacc_ref[0] + pacc_ref[1]
    c = pcnt_ref[0][:, :1] + pcnt_ref[1][:, :1]
    out_ref[...] = bm_ref[...] + s / jnp.maximum(c, 1.0)


@jax.jit
def kernel(x_user, x_item, edge_index_buys, edge_index_views, edge_index_rev):
    def full_idx(e):
        e = e.astype(jnp.int32)
        return (e[0].reshape(NS, NG, GC, CH), e[1].reshape(NS, NG, GC, CH))

    def split_idx(e):
        e = e.astype(jnp.int32)
        return (e[0].reshape(2, NS, NG, GC, CHV),
                e[1].reshape(2, NS, NG, GC, CHV))

    src_a, dst_a = full_idx(edge_index_buys)
    src_r, dst_r = full_idx(edge_index_rev)
    src_v, dst_v = split_idx(edge_index_views)

    mesh = plsc.VectorSubcoreMesh(core_axis_name="c", subcore_axis_name="s",
                                  num_cores=2, num_subcores=NS)
    f = pl.kernel(
        _body,
        out_type=(
            jax.ShapeDtypeStruct((N, D), jnp.float32),
            jax.ShapeDtypeStruct((N, D), jnp.float32),
            jax.ShapeDtypeStruct((2, N, D), jnp.float32),
            jax.ShapeDtypeStruct((2, N, CW), jnp.float32),
        ),
        mesh=mesh,
        scratch_types=[
            pltpu.VMEM_SHARED((N, D), jnp.float32),
            pltpu.VMEM_SHARED((N, CW), jnp.float32),
            pltpu.VMEM((CH, D), jnp.float32),
            pltpu.VMEM((CH, D), jnp.float32),
            pltpu.VMEM((CHV, D), jnp.float32),
            pltpu.VMEM((CHV, D), jnp.float32),
            pltpu.VMEM((GC, CH), jnp.int32),
            pltpu.VMEM((GC, CH), jnp.int32),
            pltpu.VMEM((GC, CHV), jnp.int32),
            pltpu.VMEM((GC, CHV), jnp.int32),
            pltpu.VMEM((CH, CW), jnp.float32),
            pltpu.VMEM((CH, CW), jnp.float32),
            pltpu.VMEM((CHV, CW), jnp.float32),
            pltpu.SemaphoreType.DMA,
            pltpu.SemaphoreType.DMA,
            pltpu.SemaphoreType.DMA,
            pltpu.SemaphoreType.DMA,
            pltpu.SemaphoreType.DMA,
        ],
        compiler_params=pltpu.CompilerParams(use_tc_tiling_on_sc=False),
        name="hetero_routing_sc",
    )
    out_user, buys_mean, pacc, pcnt = f(x_user, x_item, src_a, dst_a,
                                        src_r, dst_r, src_v, dst_v)

    BR = 1000
    out_item = pl.pallas_call(
        _combine_body,
        grid=(N // BR,),
        in_specs=[
            pl.BlockSpec((BR, D), lambda i: (i, 0)),
            pl.BlockSpec((2, BR, D), lambda i: (0, i, 0)),
            pl.BlockSpec((2, BR, CW), lambda i: (0, i, 0)),
        ],
        out_specs=pl.BlockSpec((BR, D), lambda i: (i, 0)),
        out_shape=jax.ShapeDtypeStruct((N, D), jnp.float32),
        name="hetero_routing_combine",
    )(buys_mean, pacc, pcnt)

    return (out_user, out_item)


# speedup vs baseline: 1.2367x; 1.2367x over previous
import jax
import jax.numpy as jnp
from jax import lax
from jax.experimental import pallas as pl
from jax.experimental.pallas import tpu as pltpu
from jax.experimental.pallas import tpu_sc as plsc

N = 10000
D = 128
E = 160000
NS = 16
CH = 80
CHV = 40
GC = 25
NG = 5
FB = 8
CW = 16
NK = D // 16


def _body(x_user, x_item, src_a, dst_a, src_r, dst_r, src_v, dst_v,
          out_user, buys_mean, pacc, pcnt,
          acc, cnt, rows0, rows1, vrows0, vrows1,
          src_i, dst_i, vsrc_i, vdst_i, fcnt, ones, vones,
          sem0, sem1, ssem0, ssem1, osem):
    core = lax.axis_index("c")
    sub = lax.axis_index("s")
    fbase = sub * (FB * CH)

    zero16 = jnp.zeros((16,), jnp.float32)
    one16 = jnp.ones((16,), jnp.float32)

    def init_ones(r, carry):
        ones[r, pl.ds(0, CW)] = one16
        return carry

    lax.fori_loop(0, CH, init_ones, 0)

    def init_vones(r, carry):
        vones[r, pl.ds(0, CW)] = one16
        return carry

    lax.fori_loop(0, CHV, init_vones, 0)

    def conv_loop(x_hbm, src_hbm, dst_hbm, bufs, sbuf, dbuf, onesbuf,
                  core_split):
        gsems = (sem0, sem1)
        ssems = (ssem0, ssem1)

        for g in range(NG):
            if core_split:
                pltpu.sync_copy(src_hbm.at[core, sub, g], sbuf)
                pltpu.sync_copy(dst_hbm.at[core, sub, g], dbuf)
            else:
                pltpu.sync_copy(src_hbm.at[sub, g], sbuf)
                pltpu.sync_copy(dst_hbm.at[sub, g], dbuf)

            def gather(cc, b):
                return pltpu.make_async_copy(
                    x_hbm.at[sbuf.at[cc]], bufs[b], gsems[b])

            def scat(cc, b):
                return pltpu.make_async_copy(
                    bufs[b], acc.at[dbuf.at[cc]], ssems[b])

            def onescat(cc):
                return pltpu.make_async_copy(
                    onesbuf, cnt.at[dbuf.at[cc]], osem)

            def drain(cc, b):
                gather(cc, b).wait()
                pltpu.sync_copy(bufs[b], acc.at[dbuf.at[cc]], add=True)
                pltpu.async_copy(onesbuf, cnt.at[dbuf.at[cc]], osem,
                                 add=True)

            for b in range(2):
                gather(b, b).start()

            def pair(i, carry):
                for b in range(2):
                    cc = 2 * i + b
                    drain(cc, b)
                    gather(cc + 2, b).start()
                return carry

            lax.fori_loop(0, GC // 2 - 1, pair, 0)

            drain(GC - 3, 0)
            gather(GC - 1, 0).start()
            drain(GC - 2, 1)
            drain(GC - 1, 0)

            def drain_ones(i, carry):
                onescat(0).wait()
                return carry

            lax.fori_loop(0, GC, drain_ones, 0)

    def zero_slices():
        def zrow(r, carry):
            for k in range(NK):
                rows0[r, pl.ds(k * 16, 16)] = zero16
            fcnt[r, pl.ds(0, CW)] = zero16
            return carry

        lax.fori_loop(0, CH, zrow, 0)
        for c in range(FB):
            rowbase = fbase + c * CH

            @pl.when(rowbase < N)
            def _():
                pltpu.sync_copy(rows0, acc.at[pl.ds(rowbase, CH)])
                pltpu.sync_copy(fcnt, cnt.at[pl.ds(rowbase, CH)])

    def finalize(out_hbm):
        for c in range(FB):
            rowbase = fbase + c * CH

            @pl.when(rowbase < N)
            def _():
                pltpu.sync_copy(acc.at[pl.ds(rowbase, CH)], rows0)
                pltpu.sync_copy(cnt.at[pl.ds(rowbase, CH)], fcnt)

                def row_fn(r, carry):
                    cv = fcnt[r, pl.ds(0, CW)]
                    scale = 1.0 / jnp.maximum(cv, 1.0)
                    for k in range(NK):
                        rows0[r, pl.ds(k * 16, 16)] = (
                            rows0[r, pl.ds(k * 16, 16)] * scale)
                    return carry

                lax.fori_loop(0, CH, row_fn, 0)
                pltpu.sync_copy(rows0, out_hbm.at[pl.ds(rowbase, CH)])

    def dump_partials():
        for c in range(FB):
            rowbase = fbase + c * CH

            @pl.when(rowbase < N)
            def _():
                pltpu.sync_copy(acc.at[pl.ds(rowbase, CH)], rows0)
                pltpu.sync_copy(rows0, pacc.at[core, pl.ds(rowbase, CH)])
                pltpu.sync_copy(cnt.at[pl.ds(rowbase, CH)], fcnt)
                pltpu.sync_copy(fcnt, pcnt.at[core, pl.ds(rowbase, CH)])

    zero_slices()
    plsc.subcore_barrier()

    @pl.when(core == 0)
    def _():
        conv_loop(x_user, src_a, dst_a, (rows0, rows1), src_i, dst_i,
                  ones, False)

    @pl.when(core == 1)
    def _():
        conv_loop(x_item, src_r, dst_r, (rows0, rows1), src_i, dst_i,
                  ones, False)

    plsc.subcore_barrier()

    @pl.when(core == 0)
    def _():
        finalize(buys_mean)

    @pl.when(core == 1)
    def _():
        finalize(out_user)

    zero_slices()
    plsc.subcore_barrier()
    conv_loop(x_user, src_v, dst_v, (vrows0, vrows1), vsrc_i, vdst_i,
              vones, True)
    plsc.subcore_barrier()
    dump_partials()


def _combine_body(bm_ref, pacc_ref, pcnt_ref, out_ref):
    s = pacc_ref[0] + pacc_ref[1]
    c = pcnt_ref[0][:, :1] + pcnt_ref[1][:, :1]
    out_ref[...] = bm_ref[...] + s / jnp.maximum(c, 1.0)


@jax.jit
def kernel(x_user, x_item, edge_index_buys, edge_index_views, edge_index_rev):
    def full_idx(e):
        e = e.astype(jnp.int32)
        return (e[0].reshape(NS, NG, GC, CH), e[1].reshape(NS, NG, GC, CH))

    def split_idx(e):
        e = e.astype(jnp.int32)
        return (e[0].reshape(2, NS, NG, GC, CHV),
                e[1].reshape(2, NS, NG, GC, CHV))

    src_a, dst_a = full_idx(edge_index_buys)
    src_r, dst_r = full_idx(edge_index_rev)
    src_v, dst_v = split_idx(edge_index_views)

    mesh = plsc.VectorSubcoreMesh(core_axis_name="c", subcore_axis_name="s",
                                  num_cores=2, num_subcores=NS)
    f = pl.kernel(
        _body,
        out_type=(
            jax.ShapeDtypeStruct((N, D), jnp.float32),
            jax.ShapeDtypeStruct((N, D), jnp.float32),
            jax.ShapeDtypeStruct((2, N, D), jnp.float32),
            jax.ShapeDtypeStruct((2, N, CW), jnp.float32),
        ),
        mesh=mesh,
        scratch_types=[
            pltpu.VMEM_SHARED((N, D), jnp.float32),
            pltpu.VMEM_SHARED((N, CW), jnp.float32),
            pltpu.VMEM((CH, D), jnp.float32),
            pltpu.VMEM((CH, D), jnp.float32),
            pltpu.VMEM((CHV, D), jnp.float32),
            pltpu.VMEM((CHV, D), jnp.float32),
            pltpu.VMEM((GC, CH), jnp.int32),
            pltpu.VMEM((GC, CH), jnp.int32),
            pltpu.VMEM((GC, CHV), jnp.int32),
            pltpu.VMEM((GC, CHV), jnp.int32),
            pltpu.VMEM((CH, CW), jnp.float32),
            pltpu.VMEM((CH, CW), jnp.float32),
            pltpu.VMEM((CHV, CW), jnp.float32),
            pltpu.SemaphoreType.DMA,
            pltpu.SemaphoreType.DMA,
            pltpu.SemaphoreType.DMA,
            pltpu.SemaphoreType.DMA,
            pltpu.SemaphoreType.DMA,
        ],
        compiler_params=pltpu.CompilerParams(use_tc_tiling_on_sc=False),
        name="hetero_routing_sc",
    )
    out_user, buys_mean, pacc, pcnt = f(x_user, x_item, src_a, dst_a,
                                        src_r, dst_r, src_v, dst_v)

    BR = 1000
    out_item = pl.pallas_call(
        _combine_body,
        grid=(N // BR,),
        in_specs=[
            pl.BlockSpec((BR, D), lambda i: (i, 0)),
            pl.BlockSpec((2, BR, D), lambda i: (0, i, 0)),
            pl.BlockSpec((2, BR, CW), lambda i: (0, i, 0)),
        ],
        out_specs=pl.BlockSpec((BR, D), lambda i: (i, 0)),
        out_shape=jax.ShapeDtypeStruct((N, D), jnp.float32),
        name="hetero_routing_combine",
    )(buys_mean, pacc, pcnt)

    return (out_user, out_item)


# speedup vs baseline: 1.2505x; 1.0112x over previous
import jax
import jax.numpy as jnp
from jax import lax
from jax.experimental import pallas as pl
from jax.experimental.pallas import tpu as pltpu
from jax.experimental.pallas import tpu_sc as plsc

N = 10000
D = 128
E = 160000
NS = 16
CH = 80
CHV = 40
GC = 25
NG = 5
FB = 8
CW = 16
NK = D // 16


def _body(x_user, x_item, src_a, dst_a, src_r, dst_r, src_v, dst_v,
          out_user, buys_mean, pacc, pcnt,
          acc, cnt, rows0, rows1, vrows0, vrows1,
          src_i, src_i2, dst_i, vsrc_i, vdst_i, fcnt, ones,
          sem0, sem1, osem, isem):
    core = lax.axis_index("c")
    sub = lax.axis_index("s")
    fbase = sub * (FB * CH)

    zero16 = jnp.zeros((16,), jnp.float32)
    one16 = jnp.ones((16,), jnp.float32)

    def init_ones(r, carry):
        ones[r, pl.ds(0, CW)] = one16
        return carry

    lax.fori_loop(0, CH, init_ones, 0)


    def conv_loop(x_hbm, src_hbm, dst_hbm, bufs, sbuf, dbuf, onesbuf,
                  core_split):
        gsems = (sem0, sem1)

        for g in range(NG):
            if core_split:
                pltpu.sync_copy(src_hbm.at[core, sub, g], sbuf)
                pltpu.sync_copy(dst_hbm.at[core, sub, g], dbuf)
            else:
                pltpu.sync_copy(src_hbm.at[sub, g], sbuf)
                pltpu.sync_copy(dst_hbm.at[sub, g], dbuf)

            def gather(cc, b):
                return pltpu.make_async_copy(
                    x_hbm.at[sbuf.at[cc]], bufs[b], gsems[b])

            def onescat(cc):
                return pltpu.make_async_copy(
                    onesbuf, cnt.at[dbuf.at[cc]], osem)

            def drain(cc, b):
                gather(cc, b).wait()
                pltpu.sync_copy(bufs[b], acc.at[dbuf.at[cc]], add=True)
                pltpu.async_copy(onesbuf, cnt.at[dbuf.at[cc]], osem,
                                 add=True)

            for b in range(2):
                gather(b, b).start()

            def pair(i, carry):
                for b in range(2):
                    cc = 2 * i + b
                    drain(cc, b)
                    gather(cc + 2, b).start()
                return carry

            lax.fori_loop(0, GC // 2 - 1, pair, 0)

            drain(GC - 3, 0)
            gather(GC - 1, 0).start()
            drain(GC - 2, 1)
            drain(GC - 1, 0)

            def drain_ones(i, carry):
                onescat(0).wait()
                return carry

            lax.fori_loop(0, GC, drain_ones, 0)

    def conv_seamless(x_hbm, src_hbm, dst_hbm):
        gsems = (sem0, sem1)
        sbufs = (src_i, src_i2)
        bufs = (rows0, rows1)

        def stage_src(g, sb):
            return pltpu.make_async_copy(src_hbm.at[sub, g], sb, isem)

        def gather(sb, cc, b):
            return pltpu.make_async_copy(
                x_hbm.at[sb.at[cc]], bufs[b], gsems[b])

        def onescat(cc):
            return pltpu.make_async_copy(ones, cnt.at[dst_i.at[cc]], osem)

        def drain(cc, b):
            gather(src_i, cc, b).wait()
            pltpu.sync_copy(bufs[b], acc.at[dst_i.at[cc]], add=True)
            pltpu.async_copy(ones, cnt.at[dst_i.at[cc]], osem, add=True)

        s0 = stage_src(0, sbufs[0])
        s0.start()
        s0.wait()
        pltpu.sync_copy(dst_hbm.at[sub, 0], dst_i)
        gather(sbufs[0], 0, 0).start()
        gather(sbufs[0], 1, 1).start()
        stage_src(1, sbufs[1]).start()

        for g in range(NG):
            par = g % 2
            sb = sbufs[par]
            if g > 0:
                pltpu.sync_copy(dst_hbm.at[sub, g], dst_i)

            def pair(i, carry):
                for j in range(2):
                    cc = 2 * i + j
                    b = (j + par) % 2
                    drain(cc, b)
                    gather(sb, cc + 2, b).start()
                return carry

            lax.fori_loop(0, GC // 2 - 1, pair, 0)

            drain(GC - 3, par)
            gather(sb, GC - 1, par).start()
            drain(GC - 2, 1 - par)
            if g + 1 < NG:
                stage_src(g + 1, sbufs[1 - par]).wait()
                gather(sbufs[1 - par], 0, 1 - par).start()
            drain(GC - 1, par)
            if g + 1 < NG:
                gather(sbufs[1 - par], 1, par).start()

            def drain_ones(i, carry):
                onescat(0).wait()
                return carry

            lax.fori_loop(0, GC, drain_ones, 0)

            if g + 2 < NG:
                stage_src(g + 2, sbufs[par]).start()

    FH = CH // 2

    def zero_slices():
        def zrow(r, carry):
            for k in range(NK):
                rows0[r, pl.ds(k * 16, 16)] = zero16
            return carry

        lax.fori_loop(0, CH, zrow, 0)

        def zcnt(r, carry):
            fcnt[r, pl.ds(0, CW)] = zero16
            return carry

        lax.fori_loop(0, FH, zcnt, 0)
        for c in range(FB):
            rowbase = fbase + c * CH

            @pl.when(rowbase < N)
            def _():
                pltpu.sync_copy(rows0, acc.at[pl.ds(rowbase, CH)])
                for h in range(2):
                    pltpu.sync_copy(fcnt,
                                    cnt.at[pl.ds(rowbase + h * FH, FH)])

    def finalize(out_hbm):
        for c in range(FB):
            rowbase = fbase + c * CH

            @pl.when(rowbase < N)
            def _():
                pltpu.sync_copy(acc.at[pl.ds(rowbase, CH)], rows0)
                for h in range(2):
                    pltpu.sync_copy(cnt.at[pl.ds(rowbase + h * FH, FH)],
                                    fcnt)

                    def row_fn(r, carry):
                        cv = fcnt[r, pl.ds(0, CW)]
                        scale = 1.0 / jnp.maximum(cv, 1.0)
                        for k in range(NK):
                            rows0[h * FH + r, pl.ds(k * 16, 16)] = (
                                rows0[h * FH + r, pl.ds(k * 16, 16)]
                                * scale)
                        return carry

                    lax.fori_loop(0, FH, row_fn, 0)
                pltpu.sync_copy(rows0, out_hbm.at[pl.ds(rowbase, CH)])

    def dump_partials():
        for c in range(FB):
            rowbase = fbase + c * CH

            @pl.when(rowbase < N)
            def _():
                pltpu.sync_copy(acc.at[pl.ds(rowbase, CH)], rows0)
                pltpu.sync_copy(rows0, pacc.at[core, pl.ds(rowbase, CH)])
                for h in range(2):
                    pltpu.sync_copy(cnt.at[pl.ds(rowbase + h * FH, FH)],
                                    fcnt)
                    pltpu.sync_copy(
                        fcnt, pcnt.at[core, pl.ds(rowbase + h * FH, FH)])

    zero_slices()
    plsc.subcore_barrier()

    @pl.when(core == 0)
    def _():
        conv_seamless(x_user, src_a, dst_a)

    @pl.when(core == 1)
    def _():
        conv_seamless(x_item, src_r, dst_r)

    plsc.subcore_barrier()

    @pl.when(core == 0)
    def _():
        finalize(buys_mean)

    @pl.when(core == 1)
    def _():
        finalize(out_user)

    zero_slices()
    plsc.subcore_barrier()
    def fill_ones(r, carry):
        fcnt[r, pl.ds(0, CW)] = one16
        return carry

    lax.fori_loop(0, CHV, fill_ones, 0)
    conv_loop(x_user, src_v, dst_v, (vrows0, vrows1), vsrc_i, vdst_i,
              fcnt, True)
    plsc.subcore_barrier()
    dump_partials()


def _combine_body(bm_ref, pacc_ref, pcnt_ref, out_ref):
    s = pacc_ref[0] + pacc_ref[1]
    c = pcnt_ref[0][:, :1] + pcnt_ref[1][:, :1]
    out_ref[...] = bm_ref[...] + s / jnp.maximum(c, 1.0)


@jax.jit
def kernel(x_user, x_item, edge_index_buys, edge_index_views, edge_index_rev):
    def full_idx(e):
        e = e.astype(jnp.int32)
        return (e[0].reshape(NS, NG, GC, CH), e[1].reshape(NS, NG, GC, CH))

    def split_idx(e):
        e = e.astype(jnp.int32)
        return (e[0].reshape(2, NS, NG, GC, CHV),
                e[1].reshape(2, NS, NG, GC, CHV))

    src_a, dst_a = full_idx(edge_index_buys)
    src_r, dst_r = full_idx(edge_index_rev)
    src_v, dst_v = split_idx(edge_index_views)

    mesh = plsc.VectorSubcoreMesh(core_axis_name="c", subcore_axis_name="s",
                                  num_cores=2, num_subcores=NS)
    f = pl.kernel(
        _body,
        out_type=(
            jax.ShapeDtypeStruct((N, D), jnp.float32),
            jax.ShapeDtypeStruct((N, D), jnp.float32),
            jax.ShapeDtypeStruct((2, N, D), jnp.float32),
            jax.ShapeDtypeStruct((2, N, CW), jnp.float32),
        ),
        mesh=mesh,
        scratch_types=[
            pltpu.VMEM_SHARED((N, D), jnp.float32),
            pltpu.VMEM_SHARED((N, CW), jnp.float32),
            pltpu.VMEM((CH, D), jnp.float32),
            pltpu.VMEM((CH, D), jnp.float32),
            pltpu.VMEM((CHV, D), jnp.float32),
            pltpu.VMEM((CHV, D), jnp.float32),
            pltpu.VMEM((GC, CH), jnp.int32),
            pltpu.VMEM((GC, CH), jnp.int32),
            pltpu.VMEM((GC, CH), jnp.int32),
            pltpu.VMEM((GC, CHV), jnp.int32),
            pltpu.VMEM((GC, CHV), jnp.int32),
            pltpu.VMEM((CH // 2, CW), jnp.float32),
            pltpu.VMEM((CH, CW), jnp.float32),
            pltpu.SemaphoreType.DMA,
            pltpu.SemaphoreType.DMA,
            pltpu.SemaphoreType.DMA,
            pltpu.SemaphoreType.DMA,
        ],
        compiler_params=pltpu.CompilerParams(use_tc_tiling_on_sc=False),
        name="hetero_routing_sc",
    )
    out_user, buys_mean, pacc, pcnt = f(x_user, x_item, src_a, dst_a,
                                        src_r, dst_r, src_v, dst_v)

    BR = 1000
    out_item = pl.pallas_call(
        _combine_body,
        grid=(N // BR,),
        in_specs=[
            pl.BlockSpec((BR, D), lambda i: (i, 0)),
            pl.BlockSpec((2, BR, D), lambda i: (0, i, 0)),
            pl.BlockSpec((2, BR, CW), lambda i: (0, i, 0)),
        ],
        out_specs=pl.BlockSpec((BR, D), lambda i: (i, 0)),
        out_shape=jax.ShapeDtypeStruct((N, D), jnp.float32),
        name="hetero_routing_combine",
    )(buys_mean, pacc, pcnt)

    return (out_user, out_item)


# speedup vs baseline: 1.2571x; 1.0052x over previous
import jax
import jax.numpy as jnp
from jax import lax
from jax.experimental import pallas as pl
from jax.experimental.pallas import tpu as pltpu
from jax.experimental.pallas import tpu_sc as plsc

N = 10000
D = 128
E = 160000
NS = 16
CH = 80
CHV = 40
GC = 25
NG = 5
FB = 8
CW = 16
NK = D // 16


def _body(x_user, x_item, src_a, dst_a, src_r, dst_r, src_v, dst_v,
          out_user, buys_mean, pacc, pcnt,
          acc, cnt, rows0, rows1, vrows0, vrows1,
          src_i, src_i2, dst_i, vsrc_i, vdst_i, fcnt, ones,
          sem0, sem1, osem, isem):
    core = lax.axis_index("c")
    sub = lax.axis_index("s")
    fbase = sub * (FB * CH)

    zero16 = jnp.zeros((16,), jnp.float32)
    one16 = jnp.ones((16,), jnp.float32)

    def init_ones(r, carry):
        ones[r, pl.ds(0, CW)] = one16
        return carry

    lax.fori_loop(0, CH, init_ones, 0)


    def conv_loop(x_hbm, src_hbm, dst_hbm, bufs, sbuf, dbuf, onesbuf,
                  core_split):
        gsems = (sem0, sem1)

        for g in range(NG):
            if core_split:
                pltpu.sync_copy(src_hbm.at[core, sub, g], sbuf)
                pltpu.sync_copy(dst_hbm.at[core, sub, g], dbuf)
            else:
                pltpu.sync_copy(src_hbm.at[sub, g], sbuf)
                pltpu.sync_copy(dst_hbm.at[sub, g], dbuf)

            def gather(cc, b):
                return pltpu.make_async_copy(
                    x_hbm.at[sbuf.at[cc]], bufs[b], gsems[b])

            def onescat(cc):
                return pltpu.make_async_copy(
                    onesbuf, cnt.at[dbuf.at[cc]], osem)

            def drain(cc, b):
                gather(cc, b).wait()
                pltpu.sync_copy(bufs[b], acc.at[dbuf.at[cc]], add=True)
                pltpu.async_copy(onesbuf, cnt.at[dbuf.at[cc]], osem,
                                 add=True)

            for b in range(2):
                gather(b, b).start()

            def pair(i, carry):
                for b in range(2):
                    cc = 2 * i + b
                    drain(cc, b)
                    gather(cc + 2, b).start()
                return carry

            lax.fori_loop(0, GC // 2 - 1, pair, 0)

            drain(GC - 3, 0)
            gather(GC - 1, 0).start()
            drain(GC - 2, 1)
            drain(GC - 1, 0)

            def drain_ones(i, carry):
                onescat(0).wait()
                return carry

            lax.fori_loop(0, GC, drain_ones, 0)

    def conv_seamless(x_hbm, src_hbm, dst_hbm):
        gsems = (sem0, sem1)
        sbufs = (src_i, src_i2)
        bufs = (rows0, rows1)

        def stage_src(g, sb):
            return pltpu.make_async_copy(src_hbm.at[sub, g], sb, isem)

        def gather(sb, cc, b):
            return pltpu.make_async_copy(
                x_hbm.at[sb.at[cc]], bufs[b], gsems[b])

        def onescat(cc):
            return pltpu.make_async_copy(ones, cnt.at[dst_i.at[cc]], osem)

        def drain(cc, b):
            gather(src_i, cc, b).wait()
            pltpu.sync_copy(bufs[b], acc.at[dst_i.at[cc]], add=True)
            pltpu.async_copy(ones, cnt.at[dst_i.at[cc]], osem, add=True)

        s0 = stage_src(0, sbufs[0])
        s0.start()
        s0.wait()
        pltpu.sync_copy(dst_hbm.at[sub, 0], dst_i)
        gather(sbufs[0], 0, 0).start()
        gather(sbufs[0], 1, 1).start()
        stage_src(1, sbufs[1]).start()

        for g in range(NG):
            par = g % 2
            sb = sbufs[par]
            if g > 0:
                pltpu.sync_copy(dst_hbm.at[sub, g], dst_i)

            def pair(i, carry):
                for j in range(2):
                    cc = 2 * i + j
                    b = (j + par) % 2
                    drain(cc, b)
                    gather(sb, cc + 2, b).start()
                return carry

            lax.fori_loop(0, GC // 2 - 1, pair, 0)

            drain(GC - 3, par)
            gather(sb, GC - 1, par).start()
            drain(GC - 2, 1 - par)
            if g + 1 < NG:
                stage_src(g + 1, sbufs[1 - par]).wait()
                gather(sbufs[1 - par], 0, 1 - par).start()
            drain(GC - 1, par)
            if g + 1 < NG:
                gather(sbufs[1 - par], 1, par).start()

            def drain_ones(i, carry):
                onescat(0).wait()
                return carry

            lax.fori_loop(0, GC, drain_ones, 0)

            if g + 2 < NG:
                stage_src(g + 2, sbufs[par]).start()

    FH = CH // 2

    def zero_slices():
        def zrow(r, carry):
            for k in range(NK):
                rows0[r, pl.ds(k * 16, 16)] = zero16
            return carry

        lax.fori_loop(0, CH, zrow, 0)

        def zcnt(r, carry):
            fcnt[r, pl.ds(0, CW)] = zero16
            return carry

        lax.fori_loop(0, FH, zcnt, 0)
        for c in range(FB):
            rowbase = fbase + c * CH

            @pl.when(rowbase < N)
            def _():
                pltpu.sync_copy(rows0, acc.at[pl.ds(rowbase, CH)])
                for h in range(2):
                    pltpu.sync_copy(fcnt,
                                    cnt.at[pl.ds(rowbase + h * FH, FH)])

    def finalize(out_hbm):
        for c in range(FB):
            rowbase = fbase + c * CH

            @pl.when(rowbase < N)
            def _():
                pltpu.sync_copy(acc.at[pl.ds(rowbase, CH)], rows0)
                for h in range(2):
                    pltpu.sync_copy(cnt.at[pl.ds(rowbase + h * FH, FH)],
                                    fcnt)

                    def row_fn(r, carry):
                        cv = fcnt[r, pl.ds(0, CW)]
                        scale = 1.0 / jnp.maximum(cv, 1.0)
                        for k in range(NK):
                            rows0[h * FH + r, pl.ds(k * 16, 16)] = (
                                rows0[h * FH + r, pl.ds(k * 16, 16)]
                                * scale)
                        return carry

                    lax.fori_loop(0, FH, row_fn, 0)
                pltpu.sync_copy(rows0, out_hbm.at[pl.ds(rowbase, CH)])

    def dump_partials():
        for c in range(FB):
            rowbase = fbase + c * CH

            @pl.when(rowbase < N)
            def _():
                pltpu.sync_copy(acc.at[pl.ds(rowbase, CH)], rows0)
                pltpu.sync_copy(rows0, pacc.at[core, pl.ds(rowbase, CH)])
                for h in range(2):
                    pltpu.sync_copy(cnt.at[pl.ds(rowbase + h * FH, FH)],
                                    fcnt)
                    pltpu.sync_copy(
                        fcnt, pcnt.at[core, pl.ds(rowbase + h * FH, FH)])

    zero_slices()
    plsc.subcore_barrier()

    @pl.when(core == 0)
    def _():
        conv_seamless(x_user, src_a, dst_a)

    @pl.when(core == 1)
    def _():
        conv_seamless(x_item, src_r, dst_r)

    plsc.subcore_barrier()

    @pl.when(core == 0)
    def _():
        finalize(buys_mean)

    @pl.when(core == 1)
    def _():
        finalize(out_user)

    zero_slices()
    plsc.subcore_barrier()
    def fill_ones(r, carry):
        fcnt[r, pl.ds(0, CW)] = one16
        return carry

    lax.fori_loop(0, CHV, fill_ones, 0)
    conv_loop(x_user, src_v, dst_v, (vrows0, vrows1), vsrc_i, vdst_i,
              fcnt, True)
    plsc.subcore_barrier()
    dump_partials()


def _combine_body(bm_ref, pacc_ref, pcnt_ref, out_ref):
    s = pacc_ref[0] + pacc_ref[1]
    c = pcnt_ref[0][:, :1] + pcnt_ref[1][:, :1]
    out_ref[...] = bm_ref[...] + s / jnp.maximum(c, 1.0)


@jax.jit
def kernel(x_user, x_item, edge_index_buys, edge_index_views, edge_index_rev):
    def full_idx(e):
        e = e.astype(jnp.int32)
        return (e[0].reshape(NS, NG, GC, CH), e[1].reshape(NS, NG, GC, CH))

    def split_idx(e):
        e = e.astype(jnp.int32)
        return (e[0].reshape(2, NS, NG, GC, CHV),
                e[1].reshape(2, NS, NG, GC, CHV))

    src_a, dst_a = full_idx(edge_index_buys)
    src_r, dst_r = full_idx(edge_index_rev)
    src_v, dst_v = split_idx(edge_index_views)

    mesh = plsc.VectorSubcoreMesh(core_axis_name="c", subcore_axis_name="s",
                                  num_cores=2, num_subcores=NS)
    f = pl.kernel(
        _body,
        out_type=(
            jax.ShapeDtypeStruct((N, D), jnp.float32),
            jax.ShapeDtypeStruct((N, D), jnp.float32),
            jax.ShapeDtypeStruct((2, N, D), jnp.float32),
            jax.ShapeDtypeStruct((2, N, CW), jnp.float32),
        ),
        mesh=mesh,
        scratch_types=[
            pltpu.VMEM_SHARED((N, D), jnp.float32),
            pltpu.VMEM_SHARED((N, CW), jnp.float32),
            pltpu.VMEM((CH, D), jnp.float32),
            pltpu.VMEM((CH, D), jnp.float32),
            pltpu.VMEM((CHV, D), jnp.float32),
            pltpu.VMEM((CHV, D), jnp.float32),
            pltpu.VMEM((GC, CH), jnp.int32),
            pltpu.VMEM((GC, CH), jnp.int32),
            pltpu.VMEM((GC, CH), jnp.int32),
            pltpu.VMEM((GC, CHV), jnp.int32),
            pltpu.VMEM((GC, CHV), jnp.int32),
            pltpu.VMEM((CH // 2, CW), jnp.float32),
            pltpu.VMEM((CH, CW), jnp.float32),
            pltpu.SemaphoreType.DMA,
            pltpu.SemaphoreType.DMA,
            pltpu.SemaphoreType.DMA,
            pltpu.SemaphoreType.DMA,
        ],
        compiler_params=pltpu.CompilerParams(use_tc_tiling_on_sc=False),
        name="hetero_routing_sc",
    )
    out_user, buys_mean, pacc, pcnt = f(x_user, x_item, src_a, dst_a,
                                        src_r, dst_r, src_v, dst_v)

    BR = 2000
    out_item = pl.pallas_call(
        _combine_body,
        grid=(N // BR,),
        in_specs=[
            pl.BlockSpec((BR, D), lambda i: (i, 0)),
            pl.BlockSpec((2, BR, D), lambda i: (0, i, 0)),
            pl.BlockSpec((2, BR, CW), lambda i: (0, i, 0)),
        ],
        out_specs=pl.BlockSpec((BR, D), lambda i: (i, 0)),
        out_shape=jax.ShapeDtypeStruct((N, D), jnp.float32),
        name="hetero_routing_combine",
    )(buys_mean, pacc, pcnt)

    return (out_user, out_item)


# speedup vs baseline: 1.2815x; 1.0194x over previous
import jax
import jax.numpy as jnp
from jax import lax
from jax.experimental import pallas as pl
from jax.experimental.pallas import tpu as pltpu
from jax.experimental.pallas import tpu_sc as plsc

N = 10000
D = 128
E = 160000
NS = 16
CH = 80
CHV = 40
GC = 25
NG = 5
FB = 8
CW = 16
NK = D // 16


def _body(x_user, x_item, src_a, dst_a, src_r, dst_r, src_v, dst_v,
          out_user, buys_mean, pacc, pcnt,
          acc, cnt, rows0, rows1, vrows0, vrows1,
          src_i, src_i2, dst_i, vsrc_i, vdst_i, fcnt, ones,
          sem0, sem1, osem, isem):
    core = lax.axis_index("c")
    sub = lax.axis_index("s")
    fbase = sub * (FB * CH)

    zero16 = jnp.zeros((16,), jnp.float32)
    one16 = jnp.ones((16,), jnp.float32)

    def init_ones(r, carry):
        ones[r, pl.ds(0, CW)] = one16
        return carry

    lax.fori_loop(0, CH, init_ones, 0)


    def conv_loop(x_hbm, src_hbm, dst_hbm, bufs, sbuf, dbuf, onesbuf,
                  core_split):
        gsems = (sem0, sem1)

        for g in range(NG):
            if core_split:
                pltpu.sync_copy(src_hbm.at[core, sub, g], sbuf)
                pltpu.sync_copy(dst_hbm.at[core, sub, g], dbuf)
            else:
                pltpu.sync_copy(src_hbm.at[sub, g], sbuf)
                pltpu.sync_copy(dst_hbm.at[sub, g], dbuf)

            def gather(cc, b):
                return pltpu.make_async_copy(
                    x_hbm.at[sbuf.at[cc]], bufs[b], gsems[b])

            def onescat(cc):
                return pltpu.make_async_copy(
                    onesbuf, cnt.at[dbuf.at[cc]], osem)

            def drain(cc, b):
                gather(cc, b).wait()
                pltpu.sync_copy(bufs[b], acc.at[dbuf.at[cc]], add=True)
                pltpu.async_copy(onesbuf, cnt.at[dbuf.at[cc]], osem,
                                 add=True)

            for b in range(2):
                gather(b, b).start()

            def pair(i, carry):
                for b in range(2):
                    cc = 2 * i + b
                    drain(cc, b)
                    gather(cc + 2, b).start()
                return carry

            lax.fori_loop(0, GC // 2 - 1, pair, 0)

            drain(GC - 3, 0)
            gather(GC - 1, 0).start()
            drain(GC - 2, 1)
            drain(GC - 1, 0)

            def drain_ones(i, carry):
                onescat(0).wait()
                return carry

            lax.fori_loop(0, GC, drain_ones, 0)

    def conv_seamless(x_hbm, src_hbm, dst_hbm):
        gsems = (sem0, sem1)
        sbufs = (src_i, src_i2)
        bufs = (rows0, rows1)

        def stage_src(g, sb):
            return pltpu.make_async_copy(src_hbm.at[sub, g], sb, isem)

        def gather(sb, cc, b):
            return pltpu.make_async_copy(
                x_hbm.at[sb.at[cc]], bufs[b], gsems[b])

        def onescat(cc):
            return pltpu.make_async_copy(ones, cnt.at[dst_i.at[cc]], osem)

        def drain(cc, b):
            gather(src_i, cc, b).wait()
            pltpu.sync_copy(bufs[b], acc.at[dst_i.at[cc]], add=True)
            pltpu.async_copy(ones, cnt.at[dst_i.at[cc]], osem, add=True)

        s0 = stage_src(0, sbufs[0])
        s0.start()
        s0.wait()
        pltpu.sync_copy(dst_hbm.at[sub, 0], dst_i)
        gather(sbufs[0], 0, 0).start()
        gather(sbufs[0], 1, 1).start()
        stage_src(1, sbufs[1]).start()

        for g in range(NG):
            par = g % 2
            sb = sbufs[par]
            if g > 0:
                pltpu.sync_copy(dst_hbm.at[sub, g], dst_i)

            def pair(i, carry):
                for j in range(2):
                    cc = 2 * i + j
                    b = (j + par) % 2
                    drain(cc, b)
                    gather(sb, cc + 2, b).start()
                return carry

            lax.fori_loop(0, GC // 2 - 1, pair, 0)

            drain(GC - 3, par)
            gather(sb, GC - 1, par).start()
            drain(GC - 2, 1 - par)
            if g + 1 < NG:
                stage_src(g + 1, sbufs[1 - par]).wait()
                gather(sbufs[1 - par], 0, 1 - par).start()
            drain(GC - 1, par)
            if g + 1 < NG:
                gather(sbufs[1 - par], 1, par).start()

            def drain_ones(i, carry):
                onescat(0).wait()
                return carry

            lax.fori_loop(0, GC, drain_ones, 0)

            if g + 2 < NG:
                stage_src(g + 2, sbufs[par]).start()

    FH = CH // 2

    def zero_slices():
        def zrow(r, carry):
            for k in range(NK):
                rows0[r, pl.ds(k * 16, 16)] = zero16
            return carry

        lax.fori_loop(0, CH, zrow, 0)

        def zcnt(r, carry):
            fcnt[r, pl.ds(0, CW)] = zero16
            return carry

        lax.fori_loop(0, FH, zcnt, 0)
        for c in range(FB):
            rowbase = fbase + c * CH

            @pl.when(rowbase < N)
            def _():
                pltpu.sync_copy(rows0, acc.at[pl.ds(rowbase, CH)])
                for h in range(2):
                    pltpu.sync_copy(fcnt,
                                    cnt.at[pl.ds(rowbase + h * FH, FH)])

    def ldblock(c, b):
        return pltpu.make_async_copy(
            acc.at[pl.ds(fbase + c * CH, CH)], (rows0, rows1)[b],
            (sem0, sem1)[b])

    def finalize(out_hbm):
        ldblock(0, 0).start()
        for c in range(FB):
            rowbase = fbase + c * CH
            buf = (rows0, rows1)[c % 2]

            @pl.when(rowbase < N)
            def _():
                if c + 1 < FB:
                    @pl.when(rowbase + CH < N)
                    def _():
                        ldblock(c + 1, (c + 1) % 2).start()
                ldblock(c, c % 2).wait()
                for h in range(2):
                    pltpu.sync_copy(cnt.at[pl.ds(rowbase + h * FH, FH)],
                                    fcnt)

                    def row_fn(r, carry):
                        cv = fcnt[r, pl.ds(0, CW)]
                        scale = 1.0 / jnp.maximum(cv, 1.0)
                        for k in range(NK):
                            buf[h * FH + r, pl.ds(k * 16, 16)] = (
                                buf[h * FH + r, pl.ds(k * 16, 16)]
                                * scale)
                        return carry

                    lax.fori_loop(0, FH, row_fn, 0)
                pltpu.sync_copy(buf, out_hbm.at[pl.ds(rowbase, CH)])

    def dump_partials():
        ldblock(0, 0).start()
        for c in range(FB):
            rowbase = fbase + c * CH
            buf = (rows0, rows1)[c % 2]

            @pl.when(rowbase < N)
            def _():
                if c + 1 < FB:
                    @pl.when(rowbase + CH < N)
                    def _():
                        ldblock(c + 1, (c + 1) % 2).start()
                ldblock(c, c % 2).wait()
                pltpu.sync_copy(buf, pacc.at[core, pl.ds(rowbase, CH)])
                for h in range(2):
                    pltpu.sync_copy(cnt.at[pl.ds(rowbase + h * FH, FH)],
                                    fcnt)
                    pltpu.sync_copy(
                        fcnt, pcnt.at[core, pl.ds(rowbase + h * FH, FH)])

    zero_slices()
    plsc.subcore_barrier()

    @pl.when(core == 0)
    def _():
        conv_seamless(x_user, src_a, dst_a)

    @pl.when(core == 1)
    def _():
        conv_seamless(x_item, src_r, dst_r)

    plsc.subcore_barrier()

    @pl.when(core == 0)
    def _():
        finalize(buys_mean)

    @pl.when(core == 1)
    def _():
        finalize(out_user)

    zero_slices()
    plsc.subcore_barrier()
    def fill_ones(r, carry):
        fcnt[r, pl.ds(0, CW)] = one16
        return carry

    lax.fori_loop(0, CHV, fill_ones, 0)
    conv_loop(x_user, src_v, dst_v, (vrows0, vrows1), vsrc_i, vdst_i,
              fcnt, True)
    plsc.subcore_barrier()
    dump_partials()


def _combine_body(bm_ref, pacc_ref, pcnt_ref, out_ref):
    s = pacc_ref[0] + pacc_ref[1]
    c = pcnt_ref[0][:, :1] + pcnt_ref[1][:, :1]
    out_ref[...] = bm_ref[...] + s / jnp.maximum(c, 1.0)


@jax.jit
def kernel(x_user, x_item, edge_index_buys, edge_index_views, edge_index_rev):
    def full_idx(e):
        e = e.astype(jnp.int32)
        return (e[0].reshape(NS, NG, GC, CH), e[1].reshape(NS, NG, GC, CH))

    def split_idx(e):
        e = e.astype(jnp.int32)
        return (e[0].reshape(2, NS, NG, GC, CHV),
                e[1].reshape(2, NS, NG, GC, CHV))

    src_a, dst_a = full_idx(edge_index_buys)
    src_r, dst_r = full_idx(edge_index_rev)
    src_v, dst_v = split_idx(edge_index_views)

    mesh = plsc.VectorSubcoreMesh(core_axis_name="c", subcore_axis_name="s",
                                  num_cores=2, num_subcores=NS)
    f = pl.kernel(
        _body,
        out_type=(
            jax.ShapeDtypeStruct((N, D), jnp.float32),
            jax.ShapeDtypeStruct((N, D), jnp.float32),
            jax.ShapeDtypeStruct((2, N, D), jnp.float32),
            jax.ShapeDtypeStruct((2, N, CW), jnp.float32),
        ),
        mesh=mesh,
        scratch_types=[
            pltpu.VMEM_SHARED((N, D), jnp.float32),
            pltpu.VMEM_SHARED((N, CW), jnp.float32),
            pltpu.VMEM((CH, D), jnp.float32),
            pltpu.VMEM((CH, D), jnp.float32),
            pltpu.VMEM((CHV, D), jnp.float32),
            pltpu.VMEM((CHV, D), jnp.float32),
            pltpu.VMEM((GC, CH), jnp.int32),
            pltpu.VMEM((GC, CH), jnp.int32),
            pltpu.VMEM((GC, CH), jnp.int32),
            pltpu.VMEM((GC, CHV), jnp.int32),
            pltpu.VMEM((GC, CHV), jnp.int32),
            pltpu.VMEM((CH // 2, CW), jnp.float32),
            pltpu.VMEM((CH, CW), jnp.float32),
            pltpu.SemaphoreType.DMA,
            pltpu.SemaphoreType.DMA,
            pltpu.SemaphoreType.DMA,
            pltpu.SemaphoreType.DMA,
        ],
        compiler_params=pltpu.CompilerParams(use_tc_tiling_on_sc=False),
        name="hetero_routing_sc",
    )
    out_user, buys_mean, pacc, pcnt = f(x_user, x_item, src_a, dst_a,
                                        src_r, dst_r, src_v, dst_v)

    BR = 2000
    out_item = pl.pallas_call(
        _combine_body,
        grid=(N // BR,),
        in_specs=[
            pl.BlockSpec((BR, D), lambda i: (i, 0)),
            pl.BlockSpec((2, BR, D), lambda i: (0, i, 0)),
            pl.BlockSpec((2, BR, CW), lambda i: (0, i, 0)),
        ],
        out_specs=pl.BlockSpec((BR, D), lambda i: (i, 0)),
        out_shape=jax.ShapeDtypeStruct((N, D), jnp.float32),
        name="hetero_routing_combine",
    )(buys_mean, pacc, pcnt)

    return (out_user, out_item)


# speedup vs baseline: 1.2926x; 1.0087x over previous
import jax
import jax.numpy as jnp
from jax import lax
from jax.experimental import pallas as pl
from jax.experimental.pallas import tpu as pltpu
from jax.experimental.pallas import tpu_sc as plsc

N = 10000
D = 128
E = 160000
NS = 16
CH = 80
CHV = 40
GC = 25
NG = 5
FB = 8
CW = 16
NK = D // 16


def _body(x_user, x_item, src_a, dst_a, src_r, dst_r, src_v, dst_v,
          out_user, buys_mean, pacc, pcnt,
          acc, cnt, rows0, rows1, vrows0, vrows1,
          src_i, src_i2, dst_i, vsrc_i, vdst_i, fcnt, ones,
          sem0, sem1, osem, isem):
    core = lax.axis_index("c")
    sub = lax.axis_index("s")
    fbase = sub * (FB * CH)

    zero16 = jnp.zeros((16,), jnp.float32)
    one16 = jnp.ones((16,), jnp.float32)

    def init_ones(r, carry):
        ones[r, pl.ds(0, CW)] = one16
        return carry

    lax.fori_loop(0, CH, init_ones, 0)


    def conv_loop(x_hbm, src_hbm, dst_hbm, bufs, sbuf, dbuf, onesbuf,
                  core_split):
        gsems = (sem0, sem1)

        for g in range(NG):
            if core_split:
                pltpu.sync_copy(src_hbm.at[core, sub, g], sbuf)
                pltpu.sync_copy(dst_hbm.at[core, sub, g], dbuf)
            else:
                pltpu.sync_copy(src_hbm.at[sub, g], sbuf)
                pltpu.sync_copy(dst_hbm.at[sub, g], dbuf)

            def gather(cc, b):
                return pltpu.make_async_copy(
                    x_hbm.at[sbuf.at[cc]], bufs[b], gsems[b])

            def onescat(cc):
                return pltpu.make_async_copy(
                    onesbuf, cnt.at[dbuf.at[cc]], osem)

            def drain(cc, b):
                gather(cc, b).wait()
                pltpu.sync_copy(bufs[b], acc.at[dbuf.at[cc]], add=True)
                pltpu.async_copy(onesbuf, cnt.at[dbuf.at[cc]], osem,
                                 add=True)

            for b in range(2):
                gather(b, b).start()

            def pair(i, carry):
                for b in range(2):
                    cc = 2 * i + b
                    drain(cc, b)
                    gather(cc + 2, b).start()
                return carry

            lax.fori_loop(0, GC // 2 - 1, pair, 0)

            drain(GC - 3, 0)
            gather(GC - 1, 0).start()
            drain(GC - 2, 1)
            drain(GC - 1, 0)

            def drain_ones(i, carry):
                onescat(0).wait()
                return carry

            lax.fori_loop(0, GC, drain_ones, 0)

    def conv_seamless(x_hbm, src_hbm, dst_hbm):
        gsems = (sem0, sem1)
        sbufs = (src_i, src_i2)
        bufs = (rows0, rows1)

        def stage_src(g, sb):
            return pltpu.make_async_copy(src_hbm.at[sub, g], sb, isem)

        def gather(sb, cc, b):
            return pltpu.make_async_copy(
                x_hbm.at[sb.at[cc]], bufs[b], gsems[b])

        def onescat(cc):
            return pltpu.make_async_copy(ones, cnt.at[dst_i.at[cc]], osem)

        def drain(cc, b):
            gather(src_i, cc, b).wait()
            pltpu.sync_copy(bufs[b], acc.at[dst_i.at[cc]], add=True)
            pltpu.async_copy(ones, cnt.at[dst_i.at[cc]], osem, add=True)

        s0 = stage_src(0, sbufs[0])
        s0.start()
        s0.wait()
        pltpu.sync_copy(dst_hbm.at[sub, 0], dst_i)
        gather(sbufs[0], 0, 0).start()
        gather(sbufs[0], 1, 1).start()
        stage_src(1, sbufs[1]).start()

        for g in range(NG):
            par = g % 2
            sb = sbufs[par]
            if g > 0:
                pltpu.sync_copy(dst_hbm.at[sub, g], dst_i)

            def pair(i, carry):
                for j in range(2):
                    cc = 2 * i + j
                    b = (j + par) % 2
                    drain(cc, b)
                    gather(sb, cc + 2, b).start()
                return carry

            lax.fori_loop(0, GC // 2 - 1, pair, 0)

            drain(GC - 3, par)
            gather(sb, GC - 1, par).start()
            drain(GC - 2, 1 - par)
            if g + 1 < NG:
                stage_src(g + 1, sbufs[1 - par]).wait()
                gather(sbufs[1 - par], 0, 1 - par).start()
            drain(GC - 1, par)
            if g + 1 < NG:
                gather(sbufs[1 - par], 1, par).start()

            def drain_ones(i, carry):
                onescat(0).wait()
                return carry

            lax.fori_loop(0, GC, drain_ones, 0)

            if g + 2 < NG:
                stage_src(g + 2, sbufs[par]).start()

    FH = CH // 2

    def zero_slices():
        def zrow(r, carry):
            for k in range(NK):
                rows0[r, pl.ds(k * 16, 16)] = zero16
            return carry

        lax.fori_loop(0, CH, zrow, 0)

        def zcnt(r, carry):
            fcnt[r, pl.ds(0, CW)] = zero16
            return carry

        lax.fori_loop(0, FH, zcnt, 0)
        for c in range(FB):
            rowbase = fbase + c * CH

            @pl.when(rowbase < N)
            def _():
                pltpu.async_copy(rows0, acc.at[pl.ds(rowbase, CH)], sem0)
                for h in range(2):
                    pltpu.async_copy(
                        fcnt, cnt.at[pl.ds(rowbase + h * FH, FH)], sem1)

        for c in range(FB):
            rowbase = fbase + c * CH

            @pl.when(rowbase < N)
            def _():
                pltpu.make_async_copy(
                    rows0, acc.at[pl.ds(rowbase, CH)], sem0).wait()
                for h in range(2):
                    pltpu.make_async_copy(
                        fcnt, cnt.at[pl.ds(rowbase + h * FH, FH)],
                        sem1).wait()

    def ldblock(c, b):
        return pltpu.make_async_copy(
            acc.at[pl.ds(fbase + c * CH, CH)], (rows0, rows1)[b],
            (sem0, sem1)[b])

    def finalize(out_hbm):
        ldblock(0, 0).start()
        for c in range(FB):
            rowbase = fbase + c * CH
            buf = (rows0, rows1)[c % 2]

            @pl.when(rowbase < N)
            def _():
                if c + 1 < FB:
                    @pl.when(rowbase + CH < N)
                    def _():
                        ldblock(c + 1, (c + 1) % 2).start()
                ldblock(c, c % 2).wait()
                for h in range(2):
                    pltpu.sync_copy(cnt.at[pl.ds(rowbase + h * FH, FH)],
                                    fcnt)

                    def row_fn(r, carry):
                        cv = fcnt[r, pl.ds(0, CW)]
                        scale = 1.0 / jnp.maximum(cv, 1.0)
                        for k in range(NK):
                            buf[h * FH + r, pl.ds(k * 16, 16)] = (
                                buf[h * FH + r, pl.ds(k * 16, 16)]
                                * scale)
                        return carry

                    lax.fori_loop(0, FH, row_fn, 0)
                pltpu.sync_copy(buf, out_hbm.at[pl.ds(rowbase, CH)])

    def dump_partials():
        ldblock(0, 0).start()
        for c in range(FB):
            rowbase = fbase + c * CH
            buf = (rows0, rows1)[c % 2]

            @pl.when(rowbase < N)
            def _():
                if c + 1 < FB:
                    @pl.when(rowbase + CH < N)
                    def _():
                        ldblock(c + 1, (c + 1) % 2).start()
                ldblock(c, c % 2).wait()
                pltpu.sync_copy(buf, pacc.at[core, pl.ds(rowbase, CH)])
                for h in range(2):
                    pltpu.sync_copy(cnt.at[pl.ds(rowbase + h * FH, FH)],
                                    fcnt)
                    pltpu.sync_copy(
                        fcnt, pcnt.at[core, pl.ds(rowbase + h * FH, FH)])

    zero_slices()
    plsc.subcore_barrier()

    @pl.when(core == 0)
    def _():
        conv_seamless(x_user, src_a, dst_a)

    @pl.when(core == 1)
    def _():
        conv_seamless(x_item, src_r, dst_r)

    plsc.subcore_barrier()

    @pl.when(core == 0)
    def _():
        finalize(buys_mean)

    @pl.when(core == 1)
    def _():
        finalize(out_user)

    zero_slices()
    plsc.subcore_barrier()
    def fill_ones(r, carry):
        fcnt[r, pl.ds(0, CW)] = one16
        return carry

    lax.fori_loop(0, CHV, fill_ones, 0)
    conv_loop(x_user, src_v, dst_v, (vrows0, vrows1), vsrc_i, vdst_i,
              fcnt, True)
    plsc.subcore_barrier()
    dump_partials()


def _combine_body(bm_ref, pacc_ref, pcnt_ref, out_ref):
    s = pacc_ref[0] + pacc_ref[1]
    c = pcnt_ref[0][:, :1] + pcnt_ref[1][:, :1]
    out_ref[...] = bm_ref[...] + s / jnp.maximum(c, 1.0)


@jax.jit
def kernel(x_user, x_item, edge_index_buys, edge_index_views, edge_index_rev):
    def full_idx(e):
        e = e.astype(jnp.int32)
        return (e[0].reshape(NS, NG, GC, CH), e[1].reshape(NS, NG, GC, CH))

    def split_idx(e):
        e = e.astype(jnp.int32)
        return (e[0].reshape(2, NS, NG, GC, CHV),
                e[1].reshape(2, NS, NG, GC, CHV))

    src_a, dst_a = full_idx(edge_index_buys)
    src_r, dst_r = full_idx(edge_index_rev)
    src_v, dst_v = split_idx(edge_index_views)

    mesh = plsc.VectorSubcoreMesh(core_axis_name="c", subcore_axis_name="s",
                                  num_cores=2, num_subcores=NS)
    f = pl.kernel(
        _body,
        out_type=(
            jax.ShapeDtypeStruct((N, D), jnp.float32),
            jax.ShapeDtypeStruct((N, D), jnp.float32),
            jax.ShapeDtypeStruct((2, N, D), jnp.float32),
            jax.ShapeDtypeStruct((2, N, CW), jnp.float32),
        ),
        mesh=mesh,
        scratch_types=[
            pltpu.VMEM_SHARED((N, D), jnp.float32),
            pltpu.VMEM_SHARED((N, CW), jnp.float32),
            pltpu.VMEM((CH, D), jnp.float32),
            pltpu.VMEM((CH, D), jnp.float32),
            pltpu.VMEM((CHV, D), jnp.float32),
            pltpu.VMEM((CHV, D), jnp.float32),
            pltpu.VMEM((GC, CH), jnp.int32),
            pltpu.VMEM((GC, CH), jnp.int32),
            pltpu.VMEM((GC, CH), jnp.int32),
            pltpu.VMEM((GC, CHV), jnp.int32),
            pltpu.VMEM((GC, CHV), jnp.int32),
            pltpu.VMEM((CH // 2, CW), jnp.float32),
            pltpu.VMEM((CH, CW), jnp.float32),
            pltpu.SemaphoreType.DMA,
            pltpu.SemaphoreType.DMA,
            pltpu.SemaphoreType.DMA,
            pltpu.SemaphoreType.DMA,
        ],
        compiler_params=pltpu.CompilerParams(use_tc_tiling_on_sc=False),
        name="hetero_routing_sc",
    )
    out_user, buys_mean, pacc, pcnt = f(x_user, x_item, src_a, dst_a,
                                        src_r, dst_r, src_v, dst_v)

    BR = 2000
    out_item = pl.pallas_call(
        _combine_body,
        grid=(N // BR,),
        in_specs=[
            pl.BlockSpec((BR, D), lambda i: (i, 0)),
            pl.BlockSpec((2, BR, D), lambda i: (0, i, 0)),
            pl.BlockSpec((2, BR, CW), lambda i: (0, i, 0)),
        ],
        out_specs=pl.BlockSpec((BR, D), lambda i: (i, 0)),
        out_shape=jax.ShapeDtypeStruct((N, D), jnp.float32),
        name="hetero_routing_combine",
    )(buys_mean, pacc, pcnt)

    return (out_user, out_item)
